# speedup vs baseline: 832.5542x; 832.5542x over previous
import functools

import jax
import jax.numpy as jnp
from jax import lax
from jax.experimental import pallas as pl

D = 128
K = 16
V = 64
B = 512
LEVELS = 7
BC = 128


def _circuit_kernel(x_ref, logits_ref, w0, w1, w2, w3, w4, w5, w6, rw_ref,
                    out_ref):
    ws = (w0, w1, w2, w3, w4, w5, w6)
    wn = []
    for w_ref in ws:
        w = w_ref[...]
        m = jnp.max(w, axis=-1, keepdims=True)
        e = jnp.exp(w - m)
        wn.append(e / jnp.sum(e, axis=-1, keepdims=True))
    logits = logits_ref[...]
    lmax = jnp.max(logits, axis=-1, keepdims=True)
    lexp = jnp.exp(logits - lmax)
    lp = (logits - lmax) - jnp.log(jnp.sum(lexp, axis=-1, keepdims=True))
    rw = rw_ref[...]
    rm = jnp.max(rw, axis=-1, keepdims=True)
    re = jnp.exp(rw - rm)
    rwn = re / jnp.sum(re, axis=-1, keepdims=True)

    for c in range(B // BC):
        xc = x_ref[:, c * BC:(c + 1) * BC]
        oh = (xc[:, None, :] ==
              lax.broadcasted_iota(jnp.int32, (D, V, BC), 1)).astype(jnp.float32)
        mars = lax.dot_general(
            lp, oh, (((2,), (1,)), ((0,), (0,))),
            preferred_element_type=jnp.float32)
        for l in range(LEVELS):
            r = mars.shape[0] // 2
            m4 = mars.reshape(r, 2, K, BC)
            left = m4[:, 0]
            right = m4[:, 1]
            mxl = jnp.max(left, axis=1, keepdims=True)
            mxr = jnp.max(right, axis=1, keepdims=True)
            el = jnp.exp(left - mxl)
            er = jnp.exp(right - mxr)
            p = (el[:, :, None, :] * er[:, None, :, :]).reshape(r, K * K, BC)
            lin = lax.dot_general(
                wn[l], p, (((2,), (1,)), ((0,), (0,))),
                preferred_element_type=jnp.float32)
            mars = jnp.log(lin) + mxl + mxr
        m0 = mars[0]
        mx = jnp.max(m0, axis=0, keepdims=True)
        e0 = jnp.exp(m0 - mx)
        lls = jnp.log(jnp.dot(rwn, e0,
                              preferred_element_type=jnp.float32)) + mx
        out_ref[:, c * BC:(c + 1) * BC] = lls


@functools.partial(jax.jit, static_argnames=("interpret",))
def kernel(inputs, input_logits, w0, w1, w2, w3, w4, w5, w6, root_w,
           interpret=False):
    x = inputs.T
    out = pl.pallas_call(
        _circuit_kernel,
        out_shape=jax.ShapeDtypeStruct((1, B), jnp.float32),
        interpret=interpret,
    )(x, input_logits, w0, w1, w2, w3, w4, w5, w6, root_w.reshape(1, K))
    return out.reshape(B)


# speedup vs baseline: 939.6850x; 1.1287x over previous
import functools

import jax
import jax.numpy as jnp
from jax import lax
from jax.experimental import pallas as pl

D = 128
K = 16
V = 64
B = 512
LEVELS = 7
BC = 512


def _circuit_kernel(x_ref, logits_ref, w0, w1, w2, w3, w4, w5, w6, rw_ref,
                    out_ref):
    ws = (w0, w1, w2, w3, w4, w5, w6)
    wn = []
    for w_ref in ws:
        w = w_ref[...]
        m = jnp.max(w, axis=-1, keepdims=True)
        e = jnp.exp(w - m)
        wn.append(e / jnp.sum(e, axis=-1, keepdims=True))
    logits = logits_ref[...]
    lmax = jnp.max(logits, axis=-1, keepdims=True)
    lexp = jnp.exp(logits - lmax)
    lp = (logits - lmax) - jnp.log(jnp.sum(lexp, axis=-1, keepdims=True))
    rw = rw_ref[...]
    rm = jnp.max(rw, axis=-1, keepdims=True)
    re = jnp.exp(rw - rm)
    rwn = re / jnp.sum(re, axis=-1, keepdims=True)

    for c in range(B // BC):
        xc = x_ref[:, c * BC:(c + 1) * BC]
        oh = (xc[:, None, :] ==
              lax.broadcasted_iota(jnp.int32, (D, V, BC), 1)).astype(jnp.float32)
        mars = lax.dot_general(
            lp, oh, (((2,), (1,)), ((0,), (0,))),
            preferred_element_type=jnp.float32)
        for l in range(LEVELS):
            r = mars.shape[0] // 2
            m4 = mars.reshape(r, 2, K, BC)
            left = m4[:, 0]
            right = m4[:, 1]
            mxl = jnp.max(left, axis=1, keepdims=True)
            mxr = jnp.max(right, axis=1, keepdims=True)
            el = jnp.exp(left - mxl)
            er = jnp.exp(right - mxr)
            p = (el[:, :, None, :] * er[:, None, :, :]).reshape(r, K * K, BC)
            lin = lax.dot_general(
                wn[l], p, (((2,), (1,)), ((0,), (0,))),
                preferred_element_type=jnp.float32)
            mars = jnp.log(lin) + mxl + mxr
        m0 = mars[0]
        mx = jnp.max(m0, axis=0, keepdims=True)
        e0 = jnp.exp(m0 - mx)
        lls = jnp.log(jnp.dot(rwn, e0,
                              preferred_element_type=jnp.float32)) + mx
        out_ref[:, c * BC:(c + 1) * BC] = lls


@functools.partial(jax.jit, static_argnames=("interpret",))
def kernel(inputs, input_logits, w0, w1, w2, w3, w4, w5, w6, root_w,
           interpret=False):
    x = inputs.T
    out = pl.pallas_call(
        _circuit_kernel,
        out_shape=jax.ShapeDtypeStruct((1, B), jnp.float32),
        interpret=interpret,
    )(x, input_logits, w0, w1, w2, w3, w4, w5, w6, root_w.reshape(1, K))
    return out.reshape(B)
